# baseline (speedup 1.0000x reference)
import jax
import jax.numpy as jnp
from jax.experimental import pallas as pl
from jax.experimental.pallas import tpu as pltpu


_NQ = 4
_NBUF = 2


def _posemb_kernel(wr_ref, out_ref, t_ref, sems):
    g = pl.program_id(0)
    ng = pl.num_programs(0)

    @pl.when(g == 0)
    def _build_tables():
        f32 = jnp.float32
        w = wr_ref[:]
        ri = jax.lax.broadcasted_iota(jnp.int32, (128, 128), 0)
        ci = jax.lax.broadcasted_iota(jnp.int32, (128, 128), 1)

        def dot(a, b):
            return jax.lax.dot(a, b, preferred_element_type=f32,
                               precision=jax.lax.Precision.HIGHEST)

        perm = (ri == (8 * (15 - ci // 8) + ci % 8)).astype(f32)
        flip = ((ri + ci) == 127).astype(f32)
        rev = dot(flip, dot(w, perm))
        roll8 = (ri == ((ci + 8) % 128)).astype(f32)
        gfw = dot(w, roll8)
        pick = ((ri == 127) & (ci == 0)).astype(f32)
        b0 = dot(pick, gfw)
        low = jnp.where((ri < 127) | (ci < 120), rev, b0)
        gup = jnp.concatenate([gfw[1:], jnp.zeros((1, 128), f32)], axis=0)
        high = jnp.where(ci < 120, gfw, gup)
        wext = jnp.concatenate([low, high], axis=0)

        lane256 = jax.lax.broadcasted_iota(jnp.int32, (256, 128), 1)
        for t in range(16):
            sh = (15 - t) * 8
            rollm = (ri == ((ci + sh) % 128)).astype(f32)
            rolled = dot(wext, rollm)
            rollup = jnp.concatenate(
                [rolled[1:], jnp.zeros((1, 128), f32)], axis=0)
            t_ref[t] = jnp.where(lane256 < (128 - sh), rolled, rollup)

    def copy_for(q, m):
        step = m * _NQ + q
        src = t_ref.at[:, pl.ds(127 - step, 128), :]
        dst = out_ref.at[pl.ds(step * 16, 16)]
        return pltpu.make_async_copy(src, dst, sems.at[q, m % _NBUF])

    for q in range(_NQ):
        @pl.when(g >= _NBUF)
        def _wait_oldest(q=q):
            copy_for(q, g - _NBUF).wait()

        copy_for(q, g).start(priority=q % 2)

    @pl.when(g == ng - 1)
    def _drain():
        for q in range(_NQ):
            for k in range(_NBUF):
                copy_for(q, ng - _NBUF + k).wait()


def kernel(x, W):
    bs, _, seq_len = x.shape
    num, out = W.shape
    assert seq_len == 2048 and num == 2048 and out == 8
    wr = W.reshape(128, 128)
    res = pl.pallas_call(
        _posemb_kernel,
        grid=(128 // _NQ,),
        in_specs=[pl.BlockSpec((128, 128), lambda g: (0, 0))],
        out_specs=pl.BlockSpec(memory_space=pl.ANY),
        out_shape=jax.ShapeDtypeStruct((2048, 128, 128), jnp.float32),
        scratch_shapes=[
            pltpu.VMEM((16, 256, 128), jnp.float32),
            pltpu.SemaphoreType.DMA((_NQ, _NBUF)),
        ],
    )(wr)
    emb = res.reshape(1, out, seq_len, seq_len)
    if bs > 1:
        emb = jnp.tile(emb, (bs, 1, 1, 1))
    return emb


# speedup vs baseline: 2.2859x; 2.2859x over previous
import jax
import jax.numpy as jnp
from jax.experimental import pallas as pl
from jax.experimental.pallas import tpu as pltpu


def _posemb_kernel(wr_ref, out_ref, t_ref):
    c = pl.program_id(0)
    h = pl.program_id(1)

    @pl.when((c == 0) & (h == 0))
    def _build_tables():
        f32 = jnp.float32
        w = wr_ref[:]
        ri = jax.lax.broadcasted_iota(jnp.int32, (128, 128), 0)
        ci = jax.lax.broadcasted_iota(jnp.int32, (128, 128), 1)

        def dot(a, b):
            return jax.lax.dot(a, b, preferred_element_type=f32,
                               precision=jax.lax.Precision.HIGHEST)

        perm = (ri == (8 * (15 - ci // 8) + ci % 8)).astype(f32)
        flip = ((ri + ci) == 127).astype(f32)
        rev = dot(flip, dot(w, perm))
        roll8 = (ri == ((ci + 8) % 128)).astype(f32)
        gfw = dot(w, roll8)
        pick = ((ri == 127) & (ci == 0)).astype(f32)
        b0 = dot(pick, gfw)
        low = jnp.where((ri < 127) | (ci < 120), rev, b0)
        gup = jnp.concatenate([gfw[1:], jnp.zeros((1, 128), f32)], axis=0)
        high = jnp.where(ci < 120, gfw, gup)
        wext = jnp.concatenate([low, high], axis=0)

        lane256 = jax.lax.broadcasted_iota(jnp.int32, (256, 128), 1)
        for t in range(16):
            sh = (15 - t) * 8
            rollm = (ri == ((ci + sh) % 128)).astype(f32)
            rolled = dot(wext, rollm)
            rollup = jnp.concatenate(
                [rolled[1:], jnp.zeros((1, 128), f32)], axis=0)
            t_ref[t] = jnp.where(lane256 < (128 - sh), rolled, rollup)

    r0 = 15 - h + 16 * (7 - c)
    out_ref[0, 0] = t_ref[:, pl.ds(r0, 128), :].reshape(128, 2048)


def kernel(x, W):
    bs, _, seq_len = x.shape
    num, out = W.shape
    assert seq_len == 2048 and num == 2048 and out == 8
    wr = W.reshape(128, 128)
    emb = pl.pallas_call(
        _posemb_kernel,
        grid=(8, 16),
        in_specs=[pl.BlockSpec((128, 128), lambda c, h: (0, 0))],
        out_specs=pl.BlockSpec((1, 1, 128, 2048), lambda c, h: (0, c, h, 0)),
        out_shape=jax.ShapeDtypeStruct((1, 8, 2048, 2048), jnp.float32),
        scratch_shapes=[
            pltpu.VMEM((16, 256, 128), jnp.float32),
        ],
    )(wr)
    if bs > 1:
        emb = jnp.tile(emb, (bs, 1, 1, 1))
    return emb


# speedup vs baseline: 3.0651x; 1.3409x over previous
import jax
import jax.numpy as jnp
from jax.experimental import pallas as pl
from jax.experimental.pallas import tpu as pltpu


def _posemb_kernel(wr_ref, out_ref, t_ref):
    c = pl.program_id(0)
    h = pl.program_id(1)

    @pl.when((c == 0) & (h == 0))
    def _build_tables():
        f32 = jnp.float32
        w = wr_ref[:]
        ri = jax.lax.broadcasted_iota(jnp.int32, (128, 128), 0)
        ci = jax.lax.broadcasted_iota(jnp.int32, (128, 128), 1)

        def dot(a, b):
            return jax.lax.dot(a, b, preferred_element_type=f32,
                               precision=jax.lax.Precision.HIGHEST)

        perm = (ri == (8 * (15 - ci // 8) + ci % 8)).astype(f32)
        flip = ((ri + ci) == 127).astype(f32)
        rev = dot(flip, dot(w, perm))
        roll8 = (ri == ((ci + 8) % 128)).astype(f32)
        gfw = dot(w, roll8)
        pick = ((ri == 127) & (ci == 0)).astype(f32)
        b0 = dot(pick, gfw)
        low = jnp.where((ri < 127) | (ci < 120), rev, b0)
        gup = jnp.concatenate([gfw[1:], jnp.zeros((1, 128), f32)], axis=0)
        high = jnp.where(ci < 120, gfw, gup)
        wext = jnp.concatenate([low, high], axis=0)

        lane256 = jax.lax.broadcasted_iota(jnp.int32, (256, 128), 1)
        for t in range(16):
            sh = (15 - t) * 8
            rollm = (ri == ((ci + sh) % 128)).astype(f32)
            rolled = dot(wext, rollm)
            rollup = jnp.concatenate(
                [rolled[1:], jnp.zeros((1, 128), f32)], axis=0)
            t_ref[t] = jnp.where(lane256 < (128 - sh), rolled, rollup)

    for dh in range(2):
        r0 = 15 - (2 * h + dh) + 16 * (7 - c)
        out_ref[0, 0, pl.ds(128 * dh, 128)] = (
            t_ref[:, pl.ds(r0, 128), :].reshape(128, 2048))


def kernel(x, W):
    bs, _, seq_len = x.shape
    num, out = W.shape
    assert seq_len == 2048 and num == 2048 and out == 8
    wr = W.reshape(128, 128)
    emb = pl.pallas_call(
        _posemb_kernel,
        grid=(8, 8),
        in_specs=[pl.BlockSpec((128, 128), lambda c, h: (0, 0))],
        out_specs=pl.BlockSpec((1, 1, 256, 2048), lambda c, h: (0, c, h, 0)),
        out_shape=jax.ShapeDtypeStruct((1, 8, 2048, 2048), jnp.float32),
        scratch_shapes=[
            pltpu.VMEM((16, 256, 128), jnp.float32),
        ],
    )(wr)
    if bs > 1:
        emb = jnp.tile(emb, (bs, 1, 1, 1))
    return emb


# speedup vs baseline: 3.7564x; 1.2255x over previous
import jax
import jax.numpy as jnp
from jax.experimental import pallas as pl
from jax.experimental.pallas import tpu as pltpu


def _posemb_kernel(wr_ref, out_ref, t_ref):
    c = pl.program_id(0)
    h = pl.program_id(1)

    @pl.when((c == 0) & (h == 0))
    def _build_tables():
        f32 = jnp.float32
        w = wr_ref[:]
        ri = jax.lax.broadcasted_iota(jnp.int32, (128, 128), 0)
        ci = jax.lax.broadcasted_iota(jnp.int32, (128, 128), 1)

        def dot(a, b):
            return jax.lax.dot(a, b, preferred_element_type=f32,
                               precision=jax.lax.Precision.HIGHEST)

        perm = (ri == (8 * (15 - ci // 8) + ci % 8)).astype(f32)
        flip = ((ri + ci) == 127).astype(f32)
        rev = dot(flip, dot(w, perm))
        roll8 = (ri == ((ci + 8) % 128)).astype(f32)
        gfw = dot(w, roll8)
        pick = ((ri == 127) & (ci == 0)).astype(f32)
        b0 = dot(pick, gfw)
        low = jnp.where((ri < 127) | (ci < 120), rev, b0)
        gup = jnp.concatenate([gfw[1:], jnp.zeros((1, 128), f32)], axis=0)
        high = jnp.where(ci < 120, gfw, gup)
        wext = jnp.concatenate([low, high], axis=0)

        lane256 = jax.lax.broadcasted_iota(jnp.int32, (256, 128), 1)
        for t in range(16):
            sh = (15 - t) * 8
            rollm = (ri == ((ci + sh) % 128)).astype(f32)
            rolled = dot(wext, rollm)
            rollup = jnp.concatenate(
                [rolled[1:], jnp.zeros((1, 128), f32)], axis=0)
            t_ref[t] = jnp.where(lane256 < (128 - sh), rolled, rollup)

    for dh in range(4):
        r0 = 15 - (4 * h + dh) + 16 * (7 - c)
        out_ref[0, 0, pl.ds(128 * dh, 128)] = (
            t_ref[:, pl.ds(r0, 128), :].reshape(128, 2048))


def kernel(x, W):
    bs, _, seq_len = x.shape
    num, out = W.shape
    assert seq_len == 2048 and num == 2048 and out == 8
    wr = W.reshape(128, 128)
    emb = pl.pallas_call(
        _posemb_kernel,
        grid=(8, 4),
        in_specs=[pl.BlockSpec((128, 128), lambda c, h: (0, 0))],
        out_specs=pl.BlockSpec((1, 1, 512, 2048), lambda c, h: (0, c, h, 0)),
        out_shape=jax.ShapeDtypeStruct((1, 8, 2048, 2048), jnp.float32),
        scratch_shapes=[
            pltpu.VMEM((16, 256, 128), jnp.float32),
        ],
    )(wr)
    if bs > 1:
        emb = jnp.tile(emb, (bs, 1, 1, 1))
    return emb


# speedup vs baseline: 3.8368x; 1.0214x over previous
import jax
import jax.numpy as jnp
from jax.experimental import pallas as pl
from jax.experimental.pallas import tpu as pltpu


def _posemb_kernel(wr_ref, out_ref, t_ref):
    c = pl.program_id(0)
    h = pl.program_id(1)

    @pl.when((c == 0) & (h == 0))
    def _build_tables():
        f32 = jnp.float32
        w = wr_ref[:]
        ri = jax.lax.broadcasted_iota(jnp.int32, (128, 128), 0)
        ci = jax.lax.broadcasted_iota(jnp.int32, (128, 128), 1)

        def dot(a, b):
            return jax.lax.dot(a, b, preferred_element_type=f32,
                               precision=jax.lax.Precision.HIGHEST)

        perm = (ri == (8 * (15 - ci // 8) + ci % 8)).astype(f32)
        flip = ((ri + ci) == 127).astype(f32)
        rev = dot(flip, dot(w, perm))
        roll8 = (ri == ((ci + 8) % 128)).astype(f32)
        gfw = dot(w, roll8)
        pick = ((ri == 127) & (ci == 0)).astype(f32)
        b0 = dot(pick, gfw)
        low = jnp.where((ri < 127) | (ci < 120), rev, b0)
        gup = jnp.concatenate([gfw[1:], jnp.zeros((1, 128), f32)], axis=0)
        high = jnp.where(ci < 120, gfw, gup)
        wext = jnp.concatenate([low, high], axis=0)

        lane256 = jax.lax.broadcasted_iota(jnp.int32, (256, 128), 1)
        for t in range(16):
            sh = (15 - t) * 8
            rollm = (ri == ((ci + sh) % 128)).astype(f32)
            rolled = dot(wext, rollm)
            rollup = jnp.concatenate(
                [rolled[1:], jnp.zeros((1, 128), f32)], axis=0)
            t_ref[t] = jnp.where(lane256 < (128 - sh), rolled, rollup)

    for dh in range(8):
        r0 = 15 - (8 * h + dh) + 16 * (7 - c)
        out_ref[0, 0, pl.ds(128 * dh, 128)] = (
            t_ref[:, pl.ds(r0, 128), :].reshape(128, 2048))


def kernel(x, W):
    bs, _, seq_len = x.shape
    num, out = W.shape
    assert seq_len == 2048 and num == 2048 and out == 8
    wr = W.reshape(128, 128)
    emb = pl.pallas_call(
        _posemb_kernel,
        grid=(8, 2),
        in_specs=[pl.BlockSpec((128, 128), lambda c, h: (0, 0))],
        out_specs=pl.BlockSpec((1, 1, 1024, 2048), lambda c, h: (0, c, h, 0)),
        out_shape=jax.ShapeDtypeStruct((1, 8, 2048, 2048), jnp.float32),
        scratch_shapes=[
            pltpu.VMEM((16, 256, 128), jnp.float32),
        ],
    )(wr)
    if bs > 1:
        emb = jnp.tile(emb, (bs, 1, 1, 1))
    return emb
